# baseline (speedup 1.0000x reference)
import jax
import jax.numpy as jnp
import numpy as np
from jax import lax
from jax.experimental import pallas as pl
from jax.experimental.pallas import tpu as pltpu
from jax.experimental.pallas import tpu_sc as plsc

N_LEVELS = 10
F = 4
BASE = 32
MAX_RES = 8192
LOG2_HASH = 18
HASHMAP = 2 ** LOG2_HASH
D = 3
GROWTH = np.exp((np.log(MAX_RES) - np.log(BASE)) / (N_LEVELS - 1))
SCALES = [float(BASE * (GROWTH ** l) - 1.0) for l in range(N_LEVELS)]
RES = [int(np.ceil(s)) + 1 for s in SCALES]
DENSE = [r ** D <= HASHMAP for r in RES]
N_DENSE = sum(DENSE)
PRIME1 = np.int32(np.uint32(2654435761).astype(np.int32))
PRIME2 = np.int32(805459861)

NC = 2
NS = 16
NW = NC * NS
L = 16
C = 128
NG = C // L

CORNERS = [(a, b, c) for a in (0, 1) for b in (0, 1) for c in (0, 1)]


def _body(x_hbm, tbl_hbm, scales_hbm, out_hbm, coords_v, scales_v,
          frac_a, frac_b, idx_a, idx_b, rows_a, rows_b,
          out_v, sem_a, sem_b):
    wid = lax.axis_index("s") * NC + lax.axis_index("c")
    B = out_hbm.shape[0] // 40
    P = B // NW
    n_chunks = P // C

    iota = lax.iota(jnp.int32, L)
    tri = iota * 3
    iota8 = iota * 8
    iota40 = iota * 40
    fracs = (frac_a, frac_b)
    idxs = (idx_a, idx_b)
    rows = (rows_a, rows_b)
    sems = (sem_a, sem_b)

    pltpu.sync_copy(scales_hbm, scales_v)

    def idx_stage(par, scale_vec, loff, dense, res):
        idx_v, frac_v = idxs[par], fracs[par]
        for g in range(NG):
            off = g * L
            pt3 = tri + off * 3
            x = plsc.load_gather(coords_v, [pt3])
            y = plsc.load_gather(coords_v, [pt3 + 1])
            z = plsc.load_gather(coords_v, [pt3 + 2])
            ips = []
            for d, coord in enumerate((x, y, z)):
                pos = coord * scale_vec + jnp.float32(0.5)
                ip = pos.astype(jnp.int32)
                frac_v[d, pl.ds(off, L)] = pos - ip.astype(jnp.float32)
                ips.append(ip)
            ix, iy, iz = ips
            if dense:
                hx = (ix + loff, ix + (loff + 1))
                hy = (iy * res, (iy + 1) * res)
                hz = (iz * (res * res), (iz + 1) * (res * res))
                for c, (cx, cy, cz) in enumerate(CORNERS):
                    idx_v[c, pl.ds(off, L)] = hx[cx] + hy[cy] + hz[cz]
            else:
                hx = (ix, ix + 1)
                hy = (iy * PRIME1, iy * PRIME1 + PRIME1)
                hz = (iz * PRIME2, iz * PRIME2 + PRIME2)
                for c, (cx, cy, cz) in enumerate(CORNERS):
                    idx_v[c, pl.ds(off, L)] = \
                        ((hx[cx] ^ hy[cy] ^ hz[cz]) & (HASHMAP - 1)) + loff

    def fire(par):
        idx_v, rows_v, sem = idxs[par], rows[par], sems[par]
        for c in range(8):
            pltpu.async_copy(tbl_hbm.at[idx_v.at[c]],
                             rows_v.at[pl.ds(c * C, C)], sem)

    def wait(par):
        idx_v, rows_v, sem = idxs[par], rows[par], sems[par]
        for c in range(8):
            pltpu.make_async_copy(tbl_hbm.at[idx_v.at[c]],
                                  rows_v.at[pl.ds(c * C, C)],
                                  sem).wait()

    def acc_stage(par, col4):
        rows_v, frac_v = rows[par], fracs[par]
        for g in range(NG):
            off = g * L
            fx = frac_v[0, pl.ds(off, L)]
            fy = frac_v[1, pl.ds(off, L)]
            fz = frac_v[2, pl.ds(off, L)]
            ofx = 1.0 - fx
            ofy = 1.0 - fy
            ofz = 1.0 - fz
            pxy = (ofx * ofy, ofx * fy, fx * ofy, fx * fy)
            tz = (ofz, fz)
            accs = [None] * F
            for c, (cx, cy, cz) in enumerate(CORNERS):
                rowv = iota + (c * C + off)
                w = pxy[cx * 2 + cy] * tz[cz]
                for f in range(F):
                    vals = plsc.load_gather(
                        rows_v, [rowv, jnp.full((L,), f, jnp.int32)])
                    contrib = w * vals
                    accs[f] = contrib if accs[f] is None else accs[f] + contrib
            obase = iota40 + (off * 40) + col4
            for f in range(F):
                plsc.store_scatter(out_v, [obase + f], accs[f])

    def chunk_body(k, _):
        base_pt = wid * P + k * C
        pltpu.sync_copy(x_hbm.at[pl.ds(base_pt * 3, C * 3)], coords_v)

        idx_stage(0, jnp.float32(SCALES[0]), 0 * HASHMAP, True, RES[0])
        fire(0)
        idx_stage(1, jnp.float32(SCALES[1]), 1 * HASHMAP, True, RES[1])
        fire(1)
        wait(0)
        acc_stage(0, 0)

        def hash_pair(j, _):
            l0 = N_DENSE + 2 * j
            s0 = plsc.load_gather(scales_v, [jnp.broadcast_to(l0, (L,))])
            idx_stage(0, s0, l0 * HASHMAP, False, 0)
            fire(0)
            wait(1)
            acc_stage(1, 4 * (l0 - 1))
            l1 = l0 + 1
            s1 = plsc.load_gather(scales_v, [jnp.broadcast_to(l1, (L,))])
            idx_stage(1, s1, l1 * HASHMAP, False, 0)
            fire(1)
            wait(0)
            acc_stage(0, 4 * l0)
            return 0

        lax.fori_loop(0, (N_LEVELS - N_DENSE) // 2, hash_pair, 0,
                      unroll=False)

        wait(1)
        acc_stage(1, 4 * (N_LEVELS - 1))

        pltpu.sync_copy(out_v, out_hbm.at[pl.ds(base_pt * 40, C * 40)])
        return 0

    lax.fori_loop(0, n_chunks, chunk_body, 0, unroll=False)


@jax.jit
def kernel(in_tensor, table):
    B = in_tensor.shape[0]
    x_flat = in_tensor.reshape(B * 3)
    tbl8 = jnp.pad(table.reshape(N_LEVELS * HASHMAP, F),
                   ((0, 0), (0, 2 * F - F)))
    scales = jnp.asarray(SCALES + [0.0] * (L - N_LEVELS), dtype=jnp.float32)
    mesh = plsc.VectorSubcoreMesh(
        core_axis_name="c", subcore_axis_name="s",
        num_cores=NC, num_subcores=NS)
    out_flat = pl.kernel(
        _body,
        out_type=jax.ShapeDtypeStruct((B * 40,), jnp.float32),
        mesh=mesh,
        scratch_types=[
            pltpu.VMEM((C * 3,), jnp.float32),
            pltpu.VMEM((L,), jnp.float32),
            pltpu.VMEM((D, C), jnp.float32),
            pltpu.VMEM((D, C), jnp.float32),
            pltpu.VMEM((8, C), jnp.int32),
            pltpu.VMEM((8, C), jnp.int32),
            pltpu.VMEM((8 * C, 2 * F), jnp.float32),
            pltpu.VMEM((8 * C, 2 * F), jnp.float32),
            pltpu.VMEM((C * 40,), jnp.float32),
            pltpu.SemaphoreType.DMA,
            pltpu.SemaphoreType.DMA,
        ],
        compiler_params=pltpu.CompilerParams(
            needs_layout_passes=False, use_tc_tiling_on_sc=False),
    )(x_flat, tbl8, scales)
    return out_flat.reshape(B, 40)


# speedup vs baseline: 1.1141x; 1.1141x over previous
import jax
import jax.numpy as jnp
import numpy as np
from jax import lax
from jax.experimental import pallas as pl
from jax.experimental.pallas import tpu as pltpu
from jax.experimental.pallas import tpu_sc as plsc

N_LEVELS = 10
F = 4
BASE = 32
MAX_RES = 8192
LOG2_HASH = 18
HASHMAP = 2 ** LOG2_HASH
D = 3
GROWTH = np.exp((np.log(MAX_RES) - np.log(BASE)) / (N_LEVELS - 1))
SCALES = [float(BASE * (GROWTH ** l) - 1.0) for l in range(N_LEVELS)]
RES = [int(np.ceil(s)) + 1 for s in SCALES]
DENSE = [r ** D <= HASHMAP for r in RES]
N_DENSE = sum(DENSE)
PRIME1 = np.int32(np.uint32(2654435761).astype(np.int32))
PRIME2 = np.int32(805459861)

NC = 2
NS = 16
NW = NC * NS
L = 16
C = 128
NG = C // L

CORNERS = [(a, b, c) for a in (0, 1) for b in (0, 1) for c in (0, 1)]


def _body(x_hbm, tbl_hbm, scales_hbm, out_hbm, coords_v, scales_v,
          frac_a, frac_b, idx_a, idx_b, rows_a, rows_b,
          out_v, sem_a, sem_b):
    wid = lax.axis_index("s") * NC + lax.axis_index("c")
    B = out_hbm.shape[0]
    P = B // NW
    n_chunks = P // C

    iota = lax.iota(jnp.int32, L)
    tri = iota * 3
    quadpat = lax.shift_right_logical(iota, 2)
    colpat = iota & 3
    iota8 = iota * 8
    fracs = (frac_a, frac_b)
    idxs = (idx_a, idx_b)
    rows = (rows_a, rows_b)
    sems = (sem_a, sem_b)

    pltpu.sync_copy(scales_hbm, scales_v)

    def idx_stage(par, scale_vec, loff, dense, res):
        idx_v, frac_v = idxs[par], fracs[par]
        for g in range(NG):
            off = g * L
            pt3 = tri + off * 3
            x = plsc.load_gather(coords_v, [pt3])
            y = plsc.load_gather(coords_v, [pt3 + 1])
            z = plsc.load_gather(coords_v, [pt3 + 2])
            ips = []
            for d, coord in enumerate((x, y, z)):
                pos = coord * scale_vec + jnp.float32(0.5)
                ip = pos.astype(jnp.int32)
                frac_v[d, pl.ds(off, L)] = pos - ip.astype(jnp.float32)
                ips.append(ip)
            ix, iy, iz = ips
            if dense:
                hx = (ix + loff, ix + (loff + 1))
                hy = (iy * res, (iy + 1) * res)
                hz = (iz * (res * res), (iz + 1) * (res * res))
                for c, (cx, cy, cz) in enumerate(CORNERS):
                    idx_v[c, pl.ds(off, L)] = hx[cx] + hy[cy] + hz[cz]
            else:
                hx = (ix, ix + 1)
                hy = (iy * PRIME1, iy * PRIME1 + PRIME1)
                hz = (iz * PRIME2, iz * PRIME2 + PRIME2)
                for c, (cx, cy, cz) in enumerate(CORNERS):
                    idx_v[c, pl.ds(off, L)] = \
                        ((hx[cx] ^ hy[cy] ^ hz[cz]) & (HASHMAP - 1)) + loff

    def fire(par):
        idx_v, rows_v, sem = idxs[par], rows[par], sems[par]
        for c in range(8):
            pltpu.async_copy(tbl_hbm.at[idx_v.at[c]],
                             rows_v.at[pl.ds(c * C, C)], sem)

    def wait(par):
        idx_v, rows_v, sem = idxs[par], rows[par], sems[par]
        for c in range(8):
            pltpu.make_async_copy(tbl_hbm.at[idx_v.at[c]],
                                  rows_v.at[pl.ds(c * C, C)],
                                  sem).wait()

    def acc_stage(par, col4):
        rows_v, frac_v = rows[par], fracs[par]
        colv = colpat + col4

        def acc_group(g, _):
            off = g * L
            fx = frac_v[0, pl.ds(off, L)]
            fy = frac_v[1, pl.ds(off, L)]
            fz = frac_v[2, pl.ds(off, L)]
            ofx = 1.0 - fx
            ofy = 1.0 - fy
            ofz = 1.0 - fz
            pxy = (ofx * ofy, ofx * fy, fx * ofy, fx * fy)
            tz = (ofz, fz)
            ws = [pxy[cx * 2 + cy] * tz[cz] for (cx, cy, cz) in CORNERS]

            def acc_quad(q, _):
                p4 = off + q * 4
                repl = quadpat + q * 4
                acc = None
                for c in range(8):
                    w_rep = jnp.take(ws[c], repl)
                    rowv = quadpat + (c * C + p4)
                    vals = plsc.load_gather(rows_v, [rowv, colpat])
                    contrib = w_rep * vals
                    acc = contrib if acc is None else acc + contrib
                plsc.store_scatter(out_v, [quadpat + p4, colv], acc)
                return 0

            lax.fori_loop(0, 4, acc_quad, 0, unroll=False)
            return 0

        lax.fori_loop(0, NG, acc_group, 0, unroll=False)

    def chunk_body(k, _):
        base_pt = wid * P + k * C
        pltpu.sync_copy(x_hbm.at[pl.ds(base_pt * 3, C * 3)], coords_v)

        idx_stage(0, jnp.float32(SCALES[0]), 0 * HASHMAP, True, RES[0])
        fire(0)
        idx_stage(1, jnp.float32(SCALES[1]), 1 * HASHMAP, True, RES[1])
        fire(1)
        wait(0)
        acc_stage(0, 0)

        def hash_pair(j, _):
            l0 = N_DENSE + 2 * j
            s0 = plsc.load_gather(scales_v, [jnp.broadcast_to(l0, (L,))])
            idx_stage(0, s0, l0 * HASHMAP, False, 0)
            fire(0)
            wait(1)
            acc_stage(1, 4 * (l0 - 1))
            l1 = l0 + 1
            s1 = plsc.load_gather(scales_v, [jnp.broadcast_to(l1, (L,))])
            idx_stage(1, s1, l1 * HASHMAP, False, 0)
            fire(1)
            wait(0)
            acc_stage(0, 4 * l0)
            return 0

        lax.fori_loop(0, (N_LEVELS - N_DENSE) // 2, hash_pair, 0,
                      unroll=False)

        wait(1)
        acc_stage(1, 4 * (N_LEVELS - 1))

        pltpu.sync_copy(out_v.at[pl.ds(0, C), pl.ds(0, 40)],
                        out_hbm.at[pl.ds(base_pt, C)])
        return 0

    lax.fori_loop(0, n_chunks, chunk_body, 0, unroll=False)


@jax.jit
def kernel(in_tensor, table):
    B = in_tensor.shape[0]
    x_flat = in_tensor.reshape(B * 3)
    tbl9 = jnp.pad(table.reshape(N_LEVELS * HASHMAP, F),
                   ((0, 0), (0, F)))
    scales = jnp.asarray(SCALES + [0.0] * (L - N_LEVELS), dtype=jnp.float32)
    mesh = plsc.VectorSubcoreMesh(
        core_axis_name="c", subcore_axis_name="s",
        num_cores=NC, num_subcores=NS)
    out_flat = pl.kernel(
        _body,
        out_type=jax.ShapeDtypeStruct((B, 40), jnp.float32),
        mesh=mesh,
        scratch_types=[
            pltpu.VMEM((C * 3,), jnp.float32),
            pltpu.VMEM((L,), jnp.float32),
            pltpu.VMEM((D, C), jnp.float32),
            pltpu.VMEM((D, C), jnp.float32),
            pltpu.VMEM((8, C), jnp.int32),
            pltpu.VMEM((8, C), jnp.int32),
            pltpu.VMEM((8 * C, 2 * F), jnp.float32),
            pltpu.VMEM((8 * C, 2 * F), jnp.float32),
            pltpu.VMEM((C, 41), jnp.float32),
            pltpu.SemaphoreType.DMA,
            pltpu.SemaphoreType.DMA,
        ],
        compiler_params=pltpu.CompilerParams(
            needs_layout_passes=False, use_tc_tiling_on_sc=False),
    )(x_flat, tbl9, scales)
    return out_flat


# speedup vs baseline: 1.1214x; 1.0065x over previous
import jax
import jax.numpy as jnp
import numpy as np
from jax import lax
from jax.experimental import pallas as pl
from jax.experimental.pallas import tpu as pltpu
from jax.experimental.pallas import tpu_sc as plsc

N_LEVELS = 10
F = 4
BASE = 32
MAX_RES = 8192
LOG2_HASH = 18
HASHMAP = 2 ** LOG2_HASH
D = 3
GROWTH = np.exp((np.log(MAX_RES) - np.log(BASE)) / (N_LEVELS - 1))
SCALES = [float(BASE * (GROWTH ** l) - 1.0) for l in range(N_LEVELS)]
RES = [int(np.ceil(s)) + 1 for s in SCALES]
DENSE = [r ** D <= HASHMAP for r in RES]
N_DENSE = sum(DENSE)
PRIME1 = np.int32(np.uint32(2654435761).astype(np.int32))
PRIME2 = np.int32(805459861)

NC = 2
NS = 16
NW = NC * NS
L = 16
C = 128
NG = C // L

CORNERS = [(a, b, c) for a in (0, 1) for b in (0, 1) for c in (0, 1)]


def _body(x_hbm, tbl_hbm, scales_hbm, out_hbm, coords_v, scales_v,
          frac_a, frac_b, idx_a, idx_b, rows_a, rows_b,
          out_v, sem_a, sem_b):
    wid = lax.axis_index("s") * NC + lax.axis_index("c")
    B = out_hbm.shape[0] * 128 // 40
    P = B // NW
    n_chunks = P // C

    iota = lax.iota(jnp.int32, L)
    tri = iota * 3
    quadpat = lax.shift_right_logical(iota, 2)
    colpat = iota & 3
    iota8 = iota * 8
    fracs = (frac_a, frac_b)
    idxs = (idx_a, idx_b)
    rows = (rows_a, rows_b)
    sems = (sem_a, sem_b)

    pltpu.sync_copy(scales_hbm, scales_v)

    def idx_stage(par, scale_vec, loff, dense, res):
        idx_v, frac_v = idxs[par], fracs[par]
        for g in range(NG):
            off = g * L
            pt3 = tri + off * 3
            x = plsc.load_gather(coords_v, [pt3])
            y = plsc.load_gather(coords_v, [pt3 + 1])
            z = plsc.load_gather(coords_v, [pt3 + 2])
            ips = []
            for d, coord in enumerate((x, y, z)):
                pos = coord * scale_vec + jnp.float32(0.5)
                ip = pos.astype(jnp.int32)
                frac_v[d, pl.ds(off, L)] = pos - ip.astype(jnp.float32)
                ips.append(ip)
            ix, iy, iz = ips
            if dense:
                hx = (ix + loff, ix + (loff + 1))
                hy = (iy * res, (iy + 1) * res)
                hz = (iz * (res * res), (iz + 1) * (res * res))
                for c, (cx, cy, cz) in enumerate(CORNERS):
                    idx_v[c, pl.ds(off, L)] = hx[cx] + hy[cy] + hz[cz]
            else:
                hx = (ix, ix + 1)
                hy = (iy * PRIME1, iy * PRIME1 + PRIME1)
                hz = (iz * PRIME2, iz * PRIME2 + PRIME2)
                for c, (cx, cy, cz) in enumerate(CORNERS):
                    idx_v[c, pl.ds(off, L)] = \
                        ((hx[cx] ^ hy[cy] ^ hz[cz]) & (HASHMAP - 1)) + loff

    def fire(par):
        idx_v, rows_v, sem = idxs[par], rows[par], sems[par]
        for c in range(8):
            pltpu.async_copy(tbl_hbm.at[idx_v.at[c]],
                             rows_v.at[pl.ds(c * C, C)], sem)

    def wait(par):
        idx_v, rows_v, sem = idxs[par], rows[par], sems[par]
        for c in range(8):
            pltpu.make_async_copy(tbl_hbm.at[idx_v.at[c]],
                                  rows_v.at[pl.ds(c * C, C)],
                                  sem).wait()

    def acc_stage(par, col4):
        rows_v, frac_v = rows[par], fracs[par]
        wpat = quadpat * 40 + colpat

        def acc_group(g, _):
            off = g * L
            fx = frac_v[0, pl.ds(off, L)]
            fy = frac_v[1, pl.ds(off, L)]
            fz = frac_v[2, pl.ds(off, L)]
            ofx = 1.0 - fx
            ofy = 1.0 - fy
            ofz = 1.0 - fz
            pxy = (ofx * ofy, ofx * fy, fx * ofy, fx * fy)
            tz = (ofz, fz)
            ws = [pxy[cx * 2 + cy] * tz[cz] for (cx, cy, cz) in CORNERS]

            def acc_quad(q, _):
                p4 = off + q * 4
                repl = quadpat + q * 4
                acc = None
                for c in range(8):
                    w_rep = jnp.take(ws[c], repl)
                    rowv = quadpat + (c * C + p4)
                    vals = plsc.load_gather(rows_v, [rowv, colpat])
                    contrib = w_rep * vals
                    acc = contrib if acc is None else acc + contrib
                w16 = wpat + (p4 * 40 + col4)
                plsc.store_scatter(
                    out_v, [lax.shift_right_logical(w16, 7), w16 & 127], acc)
                return 0

            lax.fori_loop(0, 4, acc_quad, 0, unroll=False)
            return 0

        lax.fori_loop(0, NG, acc_group, 0, unroll=False)

    def chunk_body(k, _):
        base_pt = wid * P + k * C
        pltpu.sync_copy(x_hbm.at[pl.ds(base_pt * 3, C * 3)], coords_v)

        idx_stage(0, jnp.float32(SCALES[0]), 0 * HASHMAP, True, RES[0])
        fire(0)
        idx_stage(1, jnp.float32(SCALES[1]), 1 * HASHMAP, True, RES[1])
        fire(1)
        wait(0)
        acc_stage(0, 0)

        def hash_pair(j, _):
            l0 = N_DENSE + 2 * j
            s0 = plsc.load_gather(scales_v, [jnp.broadcast_to(l0, (L,))])
            idx_stage(0, s0, l0 * HASHMAP, False, 0)
            fire(0)
            wait(1)
            acc_stage(1, 4 * (l0 - 1))
            l1 = l0 + 1
            s1 = plsc.load_gather(scales_v, [jnp.broadcast_to(l1, (L,))])
            idx_stage(1, s1, l1 * HASHMAP, False, 0)
            fire(1)
            wait(0)
            acc_stage(0, 4 * l0)
            return 0

        lax.fori_loop(0, (N_LEVELS - N_DENSE) // 2, hash_pair, 0,
                      unroll=False)

        wait(1)
        acc_stage(1, 4 * (N_LEVELS - 1))

        pltpu.sync_copy(out_v, out_hbm.at[pl.ds(base_pt * 40 // 128, 40)])
        return 0

    lax.fori_loop(0, n_chunks, chunk_body, 0, unroll=False)


@jax.jit
def kernel(in_tensor, table):
    B = in_tensor.shape[0]
    x_flat = in_tensor.reshape(B * 3)
    tbl9 = jnp.pad(table.reshape(N_LEVELS * HASHMAP, F),
                   ((0, 0), (0, F)))
    scales = jnp.asarray(SCALES + [0.0] * (L - N_LEVELS), dtype=jnp.float32)
    mesh = plsc.VectorSubcoreMesh(
        core_axis_name="c", subcore_axis_name="s",
        num_cores=NC, num_subcores=NS)
    out_flat = pl.kernel(
        _body,
        out_type=jax.ShapeDtypeStruct((B * 40 // 128, 128), jnp.float32),
        mesh=mesh,
        scratch_types=[
            pltpu.VMEM((C * 3,), jnp.float32),
            pltpu.VMEM((L,), jnp.float32),
            pltpu.VMEM((D, C), jnp.float32),
            pltpu.VMEM((D, C), jnp.float32),
            pltpu.VMEM((8, C), jnp.int32),
            pltpu.VMEM((8, C), jnp.int32),
            pltpu.VMEM((8 * C, 2 * F), jnp.float32),
            pltpu.VMEM((8 * C, 2 * F), jnp.float32),
            pltpu.VMEM((C * 40 // 128, 128), jnp.float32),
            pltpu.SemaphoreType.DMA,
            pltpu.SemaphoreType.DMA,
        ],
        compiler_params=pltpu.CompilerParams(
            needs_layout_passes=False, use_tc_tiling_on_sc=False),
    )(x_flat, tbl9, scales)
    return out_flat.reshape(B, 40)


# speedup vs baseline: 1.1916x; 1.0627x over previous
import jax
import jax.numpy as jnp
import numpy as np
from jax import lax
from jax.experimental import pallas as pl
from jax.experimental.pallas import tpu as pltpu
from jax.experimental.pallas import tpu_sc as plsc

N_LEVELS = 10
F = 4
BASE = 32
MAX_RES = 8192
LOG2_HASH = 18
HASHMAP = 2 ** LOG2_HASH
D = 3
GROWTH = np.exp((np.log(MAX_RES) - np.log(BASE)) / (N_LEVELS - 1))
SCALES = [float(BASE * (GROWTH ** l) - 1.0) for l in range(N_LEVELS)]
RES = [int(np.ceil(s)) + 1 for s in SCALES]
DENSE = [r ** D <= HASHMAP for r in RES]
PRIME1 = np.int32(np.uint32(2654435761).astype(np.int32))
PRIME2 = np.int32(805459861)

NC = 2
NS = 16
NW = NC * NS
L = 16
C = 128

CORNERS = [(a, b, c) for a in (0, 1) for b in (0, 1) for c in (0, 1)]


def _body(x_hbm, tbl_hbm, out_hbm, coords_v,
          frac_a, frac_b, idx_a, idx_b, lob_a, lob_b, rows_a, rows_b,
          out_v, sem_a, sem_b):
    wid = lax.axis_index("s") * NC + lax.axis_index("c")
    B = out_hbm.shape[0] // 40
    P = B // NW
    n_chunks = P // C

    iota = lax.iota(jnp.int32, L)
    tri = iota * 3
    iota40 = iota * 40
    fracs = (frac_a, frac_b)
    idxs = (idx_a, idx_b)
    lobs = (lob_a, lob_b)
    rows = (rows_a, rows_b)
    sems = (sem_a, sem_b)

    def idx_stage(l):
        par = l % 2
        idx_v, lob_v, frac_v = idxs[par], lobs[par], fracs[par]
        scale = jnp.float32(SCALES[l])
        res = RES[l]

        def idx_body(g, _):
            off = g * L
            pt3 = tri + off * 3
            x = plsc.load_gather(coords_v, [pt3])
            y = plsc.load_gather(coords_v, [pt3 + 1])
            z = plsc.load_gather(coords_v, [pt3 + 2])
            ips = []
            for d, coord in enumerate((x, y, z)):
                pos = coord * scale + jnp.float32(0.5)
                ip = pos.astype(jnp.int32)
                frac_v[d, pl.ds(off, L)] = pos - ip.astype(jnp.float32)
                ips.append(ip)
            ix, iy, iz = ips
            if DENSE[l]:
                hx = (ix, ix + 1)
                hy = (iy * res, (iy + 1) * res)
                hz = (iz * (res * res), (iz + 1) * (res * res))
                for c, (cx, cy, cz) in enumerate(CORNERS):
                    row = hx[cx] + hy[cy] + hz[cz] + l * HASHMAP
                    idx_v[c, pl.ds(off, L)] = lax.shift_right_logical(row, 1)
                    lob_v[c, pl.ds(off, L)] = lax.shift_left(row & 1, 2)
            else:
                hx = (ix, ix + 1)
                hy = (iy * PRIME1, iy * PRIME1 + PRIME1)
                hz = (iz * PRIME2, iz * PRIME2 + PRIME2)
                for c, (cx, cy, cz) in enumerate(CORNERS):
                    row = ((hx[cx] ^ hy[cy] ^ hz[cz]) & (HASHMAP - 1)) \
                        + l * HASHMAP
                    idx_v[c, pl.ds(off, L)] = lax.shift_right_logical(row, 1)
                    lob_v[c, pl.ds(off, L)] = lax.shift_left(row & 1, 2)
            return 0

        lax.fori_loop(0, C // L, idx_body, 0, unroll=False)

    def fire(l):
        par = l % 2
        idx_v, rows_v, sem = idxs[par], rows[par], sems[par]
        return [pltpu.async_copy(tbl_hbm.at[idx_v.at[c]],
                                 rows_v.at[pl.ds(c * C, C)], sem)
                for c in range(8)]

    def acc_stage(l):
        par = l % 2
        lob_v, rows_v, frac_v = lobs[par], rows[par], fracs[par]

        def acc_body(g, _):
            off = g * L
            fx = frac_v[0, pl.ds(off, L)]
            fy = frac_v[1, pl.ds(off, L)]
            fz = frac_v[2, pl.ds(off, L)]
            ofx = 1.0 - fx
            ofy = 1.0 - fy
            ofz = 1.0 - fz
            pxy = (ofx * ofy, ofx * fy, fx * ofy, fx * fy)
            tz = (ofz, fz)
            accs = [None] * F
            for c, (cx, cy, cz) in enumerate(CORNERS):
                rowv = iota + (c * C + off)
                lob16 = lob_v[c, pl.ds(off, L)]
                w = pxy[cx * 2 + cy] * tz[cz]
                for f in range(F):
                    vals = plsc.load_gather(rows_v, [rowv, lob16 + f])
                    contrib = w * vals
                    accs[f] = contrib if accs[f] is None else accs[f] + contrib
            obase = iota40 + (off * 40 + 4 * l)
            for f in range(F):
                plsc.store_scatter(out_v, [obase + f], accs[f])
            return 0

        lax.fori_loop(0, C // L, acc_body, 0, unroll=False)

    def chunk_body(k, _):
        base_pt = wid * P + k * C
        pltpu.sync_copy(x_hbm.at[pl.ds(base_pt * 3, C * 3)], coords_v)

        idx_stage(0)
        descs = fire(0)
        for l in range(N_LEVELS):
            if l + 1 < N_LEVELS:
                idx_stage(l + 1)
                next_descs = fire(l + 1)
            else:
                next_descs = None
            for dsc in descs:
                dsc.wait()
            acc_stage(l)
            descs = next_descs

        pltpu.sync_copy(out_v, out_hbm.at[pl.ds(base_pt * 40, C * 40)])
        return 0

    lax.fori_loop(0, n_chunks, chunk_body, 0, unroll=False)


@jax.jit
def kernel(in_tensor, table):
    B = in_tensor.shape[0]
    x_flat = in_tensor.reshape(B * 3)
    tbl8 = table.reshape(N_LEVELS * HASHMAP // 2, 2 * F)
    mesh = plsc.VectorSubcoreMesh(
        core_axis_name="c", subcore_axis_name="s",
        num_cores=NC, num_subcores=NS)
    out_flat = pl.kernel(
        _body,
        out_type=jax.ShapeDtypeStruct((B * 40,), jnp.float32),
        mesh=mesh,
        scratch_types=[
            pltpu.VMEM((C * 3,), jnp.float32),
            pltpu.VMEM((D, C), jnp.float32),
            pltpu.VMEM((D, C), jnp.float32),
            pltpu.VMEM((8, C), jnp.int32),
            pltpu.VMEM((8, C), jnp.int32),
            pltpu.VMEM((8, C), jnp.int32),
            pltpu.VMEM((8, C), jnp.int32),
            pltpu.VMEM((8 * C, 2 * F), jnp.float32),
            pltpu.VMEM((8 * C, 2 * F), jnp.float32),
            pltpu.VMEM((C * 40,), jnp.float32),
            pltpu.SemaphoreType.DMA,
            pltpu.SemaphoreType.DMA,
        ],
        compiler_params=pltpu.CompilerParams(
            needs_layout_passes=False, use_tc_tiling_on_sc=False),
    )(x_flat, tbl8)
    return out_flat.reshape(B, 40)
